# Initial kernel scaffold; baseline (speedup 1.0000x reference)
import jax
import jax.numpy as jnp
from jax.experimental import pallas as pl


def kernel(x, adj, weight, bias):
    raise NotImplementedError("write your pallas kernel here")


# speedup vs baseline: 6.1215x; 6.1215x over previous
import jax
import jax.numpy as jnp
from jax.experimental import pallas as pl
from jax.experimental.pallas import tpu as pltpu


def _pick_tile(dim, candidates):
    for t in candidates:
        if dim % t == 0:
            return t
    return dim


def _support_body(x_ref, w_ref, s_ref):
    s_ref[...] = jnp.dot(
        x_ref[...].astype(jnp.bfloat16),
        w_ref[...].astype(jnp.bfloat16),
        preferred_element_type=jnp.float32,
    ).astype(s_ref.dtype)


def _support_matmul(x, w, tm):
    n, fin = x.shape
    fout = w.shape[1]
    grid = (n // tm,)
    cost = pl.CostEstimate(
        flops=2 * n * fin * fout,
        transcendentals=0,
        bytes_accessed=4 * (n * fin + fin * fout) + 2 * n * fout,
    )
    return pl.pallas_call(
        _support_body,
        out_shape=jax.ShapeDtypeStruct((n, fout), jnp.bfloat16),
        grid_spec=pltpu.PrefetchScalarGridSpec(
            num_scalar_prefetch=0,
            grid=grid,
            in_specs=[
                pl.BlockSpec((tm, fin), lambda i: (i, 0)),
                pl.BlockSpec((fin, fout), lambda i: (0, 0)),
            ],
            out_specs=pl.BlockSpec((tm, fout), lambda i: (i, 0)),
        ),
        compiler_params=pltpu.CompilerParams(
            dimension_semantics=("parallel",),
            vmem_limit_bytes=100 * 1024 * 1024,
        ),
        cost_estimate=cost,
    )(x, w)


def _aggregate_body(adj_ref, s_ref, b_ref, o_ref):
    o_ref[...] = (
        jnp.dot(
            adj_ref[...].astype(jnp.bfloat16),
            s_ref[...],
            preferred_element_type=jnp.float32,
        )
        + b_ref[...]
    ).astype(o_ref.dtype)


def _aggregate_matmul(adj, support, bias2d, tm):
    n = adj.shape[0]
    fout = support.shape[1]
    grid = (n // tm,)
    cost = pl.CostEstimate(
        flops=2 * n * n * fout,
        transcendentals=0,
        bytes_accessed=4 * (n * n + n * fout + fout) + 2 * n * fout,
    )
    return pl.pallas_call(
        _aggregate_body,
        out_shape=jax.ShapeDtypeStruct((n, fout), jnp.float32),
        grid_spec=pltpu.PrefetchScalarGridSpec(
            num_scalar_prefetch=0,
            grid=grid,
            in_specs=[
                pl.BlockSpec((tm, n), lambda i: (i, 0)),
                pl.BlockSpec((n, fout), lambda i: (0, 0)),
                pl.BlockSpec((1, fout), lambda i: (0, 0)),
            ],
            out_specs=pl.BlockSpec((tm, fout), lambda i: (i, 0)),
        ),
        compiler_params=pltpu.CompilerParams(
            dimension_semantics=("parallel",),
            vmem_limit_bytes=100 * 1024 * 1024,
        ),
        cost_estimate=cost,
    )(adj, support, bias2d)


def kernel(x, adj, weight, bias):
    n, fin = x.shape
    fout = weight.shape[1]

    tm_s = _pick_tile(n, (1024, 512, 256, 128))
    tm_a = _pick_tile(n, (512, 256, 128))

    bias2d = bias.astype(jnp.float32).reshape(1, fout)
    support = _support_matmul(x.astype(jnp.float32), weight.astype(jnp.float32), tm_s)
    return _aggregate_matmul(adj.astype(jnp.float32), support, bias2d, tm_a)


# speedup vs baseline: 7.0433x; 1.1506x over previous
import jax
import jax.numpy as jnp
from jax.experimental import pallas as pl
from jax.experimental.pallas import tpu as pltpu


def _pick_tile(dim, candidates):
    for t in candidates:
        if dim % t == 0:
            return t
    return dim


def _fused_body(adj_ref, x_ref, w_ref, b_ref, o_ref):
    t = jnp.dot(
        adj_ref[...].astype(jnp.bfloat16),
        x_ref[...].astype(jnp.bfloat16),
        preferred_element_type=jnp.float32,
    )
    o_ref[...] = (
        jnp.dot(
            t.astype(jnp.bfloat16),
            w_ref[...].astype(jnp.bfloat16),
            preferred_element_type=jnp.float32,
        )
        + b_ref[...]
    ).astype(o_ref.dtype)


def kernel(x, adj, weight, bias):
    n, fin = x.shape
    fout = weight.shape[1]
    tm = _pick_tile(n, (512, 256, 128))
    grid = (n // tm,)

    bias2d = bias.astype(jnp.float32).reshape(1, fout)
    cost = pl.CostEstimate(
        flops=2 * n * n * fin + 2 * n * fin * fout,
        transcendentals=0,
        bytes_accessed=4 * (n * n + n * fin + fin * fout + fout + n * fout),
    )
    return pl.pallas_call(
        _fused_body,
        out_shape=jax.ShapeDtypeStruct((n, fout), jnp.float32),
        grid_spec=pltpu.PrefetchScalarGridSpec(
            num_scalar_prefetch=0,
            grid=grid,
            in_specs=[
                pl.BlockSpec((tm, n), lambda i: (i, 0)),
                pl.BlockSpec((n, fin), lambda i: (0, 0)),
                pl.BlockSpec((fin, fout), lambda i: (0, 0)),
                pl.BlockSpec((1, fout), lambda i: (0, 0)),
            ],
            out_specs=pl.BlockSpec((tm, fout), lambda i: (i, 0)),
        ),
        compiler_params=pltpu.CompilerParams(
            dimension_semantics=("parallel",),
            vmem_limit_bytes=100 * 1024 * 1024,
        ),
        cost_estimate=cost,
    )(adj, x.astype(jnp.float32), weight.astype(jnp.float32), bias2d)
